# Initial kernel scaffold; baseline (speedup 1.0000x reference)
import jax
import jax.numpy as jnp
from jax.experimental import pallas as pl


def kernel(x, edge_index, W, b):
    raise NotImplementedError("write your pallas kernel here")


# speedup vs baseline: 6.1612x; 6.1612x over previous
import functools

import jax
import jax.numpy as jnp
from jax import lax
from jax.experimental import pallas as pl
from jax.experimental.pallas import tpu as pltpu
from jax.experimental.pallas import tpu_sc as plsc

N_NODES = 10000
N_EDGES = 320000
D = 128

NC = 2
NS = 16
NW = NC * NS
EPW = N_EDGES // NW
CHUNK = 80
NCHUNKS = EPW // CHUNK
ROWS_PER_TILE = N_NODES // NS
ZROWS = 125


def _sc_aggregate(x, src, dst):
  mesh = plsc.VectorSubcoreMesh(core_axis_name="c", subcore_axis_name="s")

  @functools.partial(
      pl.kernel,
      out_type=[
          jax.ShapeDtypeStruct((NC, NS, ROWS_PER_TILE, D), jnp.float32),
          jax.ShapeDtypeStruct((NW, N_NODES), jnp.float32),
      ],
      mesh=mesh,
      scratch_types=[
          pltpu.VMEM((CHUNK,), jnp.int32),
          pltpu.VMEM((CHUNK,), jnp.int32),
          pltpu.VMEM((CHUNK, D), jnp.float32),
          pltpu.VMEM((N_NODES,), jnp.float32),
          pltpu.VMEM((ZROWS, D), jnp.float32),
          pltpu.VMEM_SHARED((N_NODES, D), jnp.float32),
          pltpu.SemaphoreType.DMA,
      ],
      compiler_params=pltpu.CompilerParams(needs_layout_passes=False),
  )
  def agg_kernel(x_hbm, src_hbm, dst_hbm, agg_out, deg_out,
                 sidx, didx, rows, deg_local, zblk, acc, sem):
    c = lax.axis_index("c")
    s = lax.axis_index("s")
    wid = c * NS + s

    zeros16 = jnp.zeros((16,), jnp.float32)

    def zero_zblk(i, carry):
      for g in range(D // 16):
        zblk[i, pl.ds(g * 16, 16)] = zeros16
      return carry

    lax.fori_loop(0, ZROWS, zero_zblk, 0)

    def zero_deg(i, carry):
      deg_local[pl.ds(i * 16, 16)] = zeros16
      return carry

    lax.fori_loop(0, N_NODES // 16, zero_deg, 0)

    for j in range(ROWS_PER_TILE // ZROWS):
      pltpu.sync_copy(zblk, acc.at[pl.ds(s * ROWS_PER_TILE + j * ZROWS, ZROWS)])
    plsc.subcore_barrier()

    ones16 = jnp.ones((16,), jnp.float32)

    def edge_chunk(k, carry):
      base = wid * EPW + k * CHUNK
      pltpu.sync_copy(src_hbm.at[pl.ds(base, CHUNK)], sidx)
      pltpu.sync_copy(dst_hbm.at[pl.ds(base, CHUNK)], didx)
      pltpu.async_copy(x_hbm.at[sidx], rows, sem).wait()
      pltpu.sync_copy(rows, acc.at[didx], add=True)
      for g in range(CHUNK // 16):
        idx16 = didx[pl.ds(g * 16, 16)]
        plsc.addupdate_scatter(deg_local, [idx16], ones16)
      return carry

    lax.fori_loop(0, NCHUNKS, edge_chunk, 0)

    pltpu.sync_copy(deg_local, deg_out.at[wid])
    plsc.subcore_barrier()
    pltpu.sync_copy(acc.at[pl.ds(s * ROWS_PER_TILE, ROWS_PER_TILE)],
                    agg_out.at[c, s])

  return agg_kernel(x, src, dst)


BLK = 1000


def _tc_deg_reduce(deg_part):
  def red_kernel(deg_ref, out_ref):
    out_ref[...] = jnp.sum(deg_ref[...], axis=0, keepdims=True)

  return pl.pallas_call(
      red_kernel,
      out_shape=jax.ShapeDtypeStruct((1, N_NODES), jnp.float32),
  )(deg_part)


def _tc_mlp(agg_part, deg_col, w, b2):
  grid = N_NODES // BLK

  def mlp_kernel(agg_ref, deg_ref, w_ref, b_ref, out_ref, sum_ref):
    i = pl.program_id(0)
    agg = agg_ref[0] + agg_ref[1]
    deg = deg_ref[...]
    agg = agg / jnp.maximum(deg, 1.0)
    ms = jnp.mean(agg * agg, axis=1, keepdims=True)
    h = agg / (jnp.sqrt(ms) + 1e-8)
    lin = jnp.dot(h, w_ref[...], preferred_element_type=jnp.float32) + b_ref[...]
    out_ref[...] = jnp.maximum(lin, 0.0)

    @pl.when(i == 0)
    def _init():
      sum_ref[0, 0] = 0.0

    sum_ref[0, 0] += jnp.sum(lin)

    @pl.when(i == grid - 1)
    def _finish():
      sum_ref[0, 0] = sum_ref[0, 0] / (N_NODES * D)

  return pl.pallas_call(
      mlp_kernel,
      grid=(grid,),
      in_specs=[
          pl.BlockSpec((NC, BLK, D), lambda i: (0, i, 0)),
          pl.BlockSpec((BLK, 1), lambda i: (i, 0)),
          pl.BlockSpec((D, D), lambda i: (0, 0)),
          pl.BlockSpec((1, D), lambda i: (0, 0)),
      ],
      out_specs=[
          pl.BlockSpec((BLK, D), lambda i: (i, 0)),
          pl.BlockSpec((1, 1), lambda i: (0, 0), memory_space=pltpu.SMEM),
      ],
      out_shape=[
          jax.ShapeDtypeStruct((N_NODES, D), jnp.float32),
          jax.ShapeDtypeStruct((1, 1), jnp.float32),
      ],
  )(agg_part, deg_col, w, b2)


def kernel(x, edge_index, W, b):
  src = edge_index[0]
  dst = edge_index[1]
  agg_part, deg_part = _sc_aggregate(x, src, dst)
  agg_part = agg_part.reshape(NC, N_NODES, D)
  deg_col = _tc_deg_reduce(deg_part).reshape(N_NODES, 1)
  out, sums = _tc_mlp(agg_part, deg_col, W, b.reshape(1, D))
  return out, sums.reshape(())
